# baseline (speedup 1.0000x reference)
import functools

import numpy as np
import jax
import jax.numpy as jnp
from jax import lax
from jax.experimental import pallas as pl
from jax.experimental.pallas import tpu as pltpu
from jax.experimental.pallas import tpu_sc as plsc

B_SEQ = 1024
T_SEQ = 200
D = 128
N_TOK = B_SEQ * T_SEQ
NC, NS_SUB, L = 2, 16, 16
NW = NC * NS_SUB
PER_W = N_TOK // NW
CHUNK = 64
NCHUNK = PER_W // CHUNK
NG = 10
G = 1
NSC = 2
STEP = 20
SCALE = float(np.sqrt(float(D)))

_mesh = plsc.VectorSubcoreMesh(core_axis_name="c", subcore_axis_name="s")


@functools.partial(
    pl.kernel,
    out_type=jax.ShapeDtypeStruct((N_TOK, D), jnp.float32),
    mesh=_mesh,
    scratch_types=[
        pltpu.VMEM((NCHUNK, CHUNK), jnp.int32),
        [pltpu.VMEM((CHUNK, D), jnp.float32) for _ in range(NG)],
        [pltpu.VMEM((G * CHUNK, D), jnp.float32) for _ in range(NSC)],
        [pltpu.SemaphoreType.DMA for _ in range(NG)],
        [pltpu.SemaphoreType.DMA for _ in range(NSC)],
    ],
)
def _embed(x_hbm, w_hbm, out_hbm, idx_v, gbufs, sbufs, sgs, sss):
    wid = lax.axis_index("s") * NC + lax.axis_index("c")
    base = wid * PER_W

    pltpu.sync_copy(x_hbm.at[wid], idx_v)

    def fire_gather(n, p):
        pltpu.async_copy(w_hbm.at[idx_v.at[n]], gbufs[p], sgs[p])

    def wait_gather(n, p):
        pltpu.make_async_copy(w_hbm.at[idx_v.at[n]], gbufs[p], sgs[p]).wait()

    def fire_scatter(grp, p):
        pltpu.async_copy(
            sbufs[p], out_hbm.at[pl.ds(base + grp * G * CHUNK, G * CHUNK)], sss[p]
        )

    def wait_scatter(grp, p):
        pltpu.make_async_copy(
            sbufs[p], out_hbm.at[pl.ds(base + grp * G * CHUNK, G * CHUNK)], sss[p]
        ).wait()

    def scale(gp, sp, part):
        src = gbufs[gp]
        dst = sbufs[sp]

        @plsc.parallel_loop(0, CHUNK, unroll=4)
        def _row(r):
            for c in range(D // L):
                dst[part * CHUNK + r, pl.ds(c * L, L)] = (
                    src[r, pl.ds(c * L, L)] * SCALE
                )

    for b in range(NG):
        fire_gather(b, b)

    @pl.loop(0, NCHUNK, step=STEP)
    def _grp(g):
        for b in range(STEP):
            n = g + b
            gp = b % NG
            part = b % G
            grp_slot = (b // G) % NSC
            wait_gather(n, gp)

            if part == 0:
                @pl.when(n >= G * NSC)
                def _():
                    wait_scatter(n // G - NSC, grp_slot)

            scale(gp, grp_slot, part)

            if part == G - 1:
                fire_scatter(n // G, grp_slot)

            @pl.when(n + NG < NCHUNK)
            def _():
                fire_gather(n + NG, gp)

    for m in range(NCHUNK // G - NSC, NCHUNK // G):
        wait_scatter(m, m % NSC)


def kernel(x, W):
    x_r = x.reshape(NW, NCHUNK, CHUNK).astype(jnp.int32)
    out = _embed(x_r, W)
    return out.reshape(B_SEQ, T_SEQ, D)


# speedup vs baseline: 1.0028x; 1.0028x over previous
import functools

import numpy as np
import jax
import jax.numpy as jnp
from jax import lax
from jax.experimental import pallas as pl
from jax.experimental.pallas import tpu as pltpu
from jax.experimental.pallas import tpu_sc as plsc

B_SEQ = 1024
T_SEQ = 200
D = 128
N_TOK = B_SEQ * T_SEQ
NC, NS_SUB, L = 2, 16, 16
NW = NC * NS_SUB
PER_W = N_TOK // NW
CHUNK = 64
NCHUNK = PER_W // CHUNK
NG = 10
NSC = 4
STEP = 20
SCALE = float(np.sqrt(float(D)))

_mesh = plsc.VectorSubcoreMesh(core_axis_name="c", subcore_axis_name="s")


@functools.partial(
    pl.kernel,
    out_type=jax.ShapeDtypeStruct((N_TOK, D), jnp.float32),
    mesh=_mesh,
    scratch_types=[
        pltpu.VMEM((NCHUNK, CHUNK), jnp.int32),
        [pltpu.VMEM((CHUNK, D), jnp.float32) for _ in range(NG)],
        [pltpu.VMEM((CHUNK, D), jnp.float32) for _ in range(NSC)],
        [pltpu.SemaphoreType.DMA for _ in range(NG)],
        [pltpu.SemaphoreType.DMA for _ in range(NSC)],
    ],
)
def _embed(x_hbm, w_hbm, out_hbm, idx_v, gbufs, sbufs, sgs, sss):
    wid = lax.axis_index("s") * NC + lax.axis_index("c")
    base = wid * PER_W

    pltpu.sync_copy(x_hbm.at[wid], idx_v)

    def fire_gather(n, p):
        pltpu.async_copy(w_hbm.at[idx_v.at[n]], gbufs[p], sgs[p])

    def wait_gather(n, p):
        pltpu.make_async_copy(w_hbm.at[idx_v.at[n]], gbufs[p], sgs[p]).wait()

    def fire_scatter(n, p):
        pltpu.async_copy(
            sbufs[p], out_hbm.at[pl.ds(base + n * CHUNK, CHUNK)], sss[p]
        )

    def wait_scatter(n, p):
        pltpu.make_async_copy(
            sbufs[p], out_hbm.at[pl.ds(base + n * CHUNK, CHUNK)], sss[p]
        ).wait()

    def scale(gp, sp):
        src = gbufs[gp]
        dst = sbufs[sp]

        @plsc.parallel_loop(0, CHUNK, unroll=4)
        def _row(r):
            for c in range(D // L):
                dst[r, pl.ds(c * L, L)] = src[r, pl.ds(c * L, L)] * SCALE

    for b in range(NG):
        fire_gather(b, b)

    @pl.loop(0, NCHUNK, step=STEP)
    def _grp(g):
        for b in range(STEP):
            n = g + b
            gp = b % NG
            sp = b % NSC
            wait_gather(n, gp)

            @pl.when(n >= NSC)
            def _():
                wait_scatter(n - NSC, sp)

            scale(gp, sp)
            fire_scatter(n, sp)

            @pl.when(n + NG < NCHUNK)
            def _():
                fire_gather(n + NG, gp)

    for m in range(NCHUNK - NSC, NCHUNK):
        wait_scatter(m, m % NSC)


def kernel(x, W):
    x_r = x.reshape(NW, NCHUNK, CHUNK).astype(jnp.int32)
    out = _embed(x_r, W)
    return out.reshape(B_SEQ, T_SEQ, D)


# speedup vs baseline: 1.0327x; 1.0298x over previous
import functools

import numpy as np
import jax
import jax.numpy as jnp
from jax import lax
from jax.experimental import pallas as pl
from jax.experimental.pallas import tpu as pltpu
from jax.experimental.pallas import tpu_sc as plsc

B_SEQ = 1024
T_SEQ = 200
D = 128
N_TOK = B_SEQ * T_SEQ
NC, NS_SUB, L = 2, 16, 16
NW = NC * NS_SUB
PER_W = N_TOK // NW
CHUNK = 64
NCHUNK = PER_W // CHUNK
NG = 10
NSC = 2
STEP = 10
SCALE = float(np.sqrt(float(D)))

_mesh = plsc.VectorSubcoreMesh(core_axis_name="c", subcore_axis_name="s")


@functools.partial(
    pl.kernel,
    out_type=jax.ShapeDtypeStruct((N_TOK, D), jnp.float32),
    mesh=_mesh,
    scratch_types=[
        pltpu.VMEM((NCHUNK, CHUNK), jnp.int32),
        [pltpu.VMEM((CHUNK, D), jnp.float32) for _ in range(NG)],
        [pltpu.VMEM((CHUNK, D), jnp.float32) for _ in range(NSC)],
        [pltpu.SemaphoreType.DMA for _ in range(NG)],
        [pltpu.SemaphoreType.DMA for _ in range(NSC)],
    ],
)
def _embed(x_hbm, w_hbm, out_hbm, idx_v, gbufs, sbufs, sgs, sss):
    wid = lax.axis_index("s") * NC + lax.axis_index("c")
    base = wid * PER_W

    pltpu.sync_copy(x_hbm.at[wid], idx_v)

    def fire_gather(n, p):
        pltpu.async_copy(w_hbm.at[idx_v.at[n]], gbufs[p], sgs[p])

    def wait_gather(n, p):
        pltpu.make_async_copy(w_hbm.at[idx_v.at[n]], gbufs[p], sgs[p]).wait()

    def fire_scatter(n, p):
        pltpu.async_copy(
            sbufs[p], out_hbm.at[pl.ds(base + n * CHUNK, CHUNK)], sss[p]
        )

    def wait_scatter(n, p):
        pltpu.make_async_copy(
            sbufs[p], out_hbm.at[pl.ds(base + n * CHUNK, CHUNK)], sss[p]
        ).wait()

    def scale(gp, sp):
        src = gbufs[gp]
        dst = sbufs[sp]

        @plsc.parallel_loop(0, CHUNK, unroll=4)
        def _row(r):
            for c in range(D // L):
                dst[r, pl.ds(c * L, L)] = src[r, pl.ds(c * L, L)] * SCALE

    for b in range(NG):
        fire_gather(b, b)

    @pl.loop(0, NCHUNK, step=STEP)
    def _grp(g):
        for b in range(STEP):
            n = g + b
            gp = b % NG
            sp = b % NSC
            wait_gather(n, gp)

            @pl.when(n >= NSC)
            def _():
                wait_scatter(n - NSC, sp)

            scale(gp, sp)
            fire_scatter(n, sp)

            @pl.when(n + NG < NCHUNK)
            def _():
                fire_gather(n + NG, gp)

    for m in range(NCHUNK - NSC, NCHUNK):
        wait_scatter(m, m % NSC)


def kernel(x, W):
    x_r = x.reshape(NW, NCHUNK, CHUNK).astype(jnp.int32)
    out = _embed(x_r, W)
    return out.reshape(B_SEQ, T_SEQ, D)


# speedup vs baseline: 1.0504x; 1.0171x over previous
import functools

import numpy as np
import jax
import jax.numpy as jnp
from jax import lax
from jax.experimental import pallas as pl
from jax.experimental.pallas import tpu as pltpu
from jax.experimental.pallas import tpu_sc as plsc

B_SEQ = 1024
T_SEQ = 200
D = 128
N_TOK = B_SEQ * T_SEQ
NC, NS_SUB, L = 2, 16, 16
NW = NC * NS_SUB
PER_W = N_TOK // NW
CHUNK = 80
NCHUNK = PER_W // CHUNK
NG = 8
NSC = 2
STEP = 8
SCALE = float(np.sqrt(float(D)))

_mesh = plsc.VectorSubcoreMesh(core_axis_name="c", subcore_axis_name="s")


@functools.partial(
    pl.kernel,
    out_type=jax.ShapeDtypeStruct((N_TOK, D), jnp.float32),
    mesh=_mesh,
    scratch_types=[
        pltpu.VMEM((NCHUNK, CHUNK), jnp.int32),
        [pltpu.VMEM((CHUNK, D), jnp.float32) for _ in range(NG)],
        [pltpu.VMEM((CHUNK, D), jnp.float32) for _ in range(NSC)],
        [pltpu.SemaphoreType.DMA for _ in range(NG)],
        [pltpu.SemaphoreType.DMA for _ in range(NSC)],
    ],
)
def _embed(x_hbm, w_hbm, out_hbm, idx_v, gbufs, sbufs, sgs, sss):
    wid = lax.axis_index("s") * NC + lax.axis_index("c")
    base = wid * PER_W

    pltpu.sync_copy(x_hbm.at[wid], idx_v)

    def fire_gather(n, p):
        pltpu.async_copy(w_hbm.at[idx_v.at[n]], gbufs[p], sgs[p])

    def wait_gather(n, p):
        pltpu.make_async_copy(w_hbm.at[idx_v.at[n]], gbufs[p], sgs[p]).wait()

    def fire_scatter(n, p):
        pltpu.async_copy(
            sbufs[p], out_hbm.at[pl.ds(base + n * CHUNK, CHUNK)], sss[p]
        )

    def wait_scatter(n, p):
        pltpu.make_async_copy(
            sbufs[p], out_hbm.at[pl.ds(base + n * CHUNK, CHUNK)], sss[p]
        ).wait()

    def scale(gp, sp):
        src = gbufs[gp]
        dst = sbufs[sp]

        @plsc.parallel_loop(0, CHUNK, unroll=4)
        def _row(r):
            for c in range(D // L):
                dst[r, pl.ds(c * L, L)] = src[r, pl.ds(c * L, L)] * SCALE

    for b in range(NG):
        fire_gather(b, b)

    @pl.loop(0, NCHUNK, step=STEP)
    def _grp(g):
        for b in range(STEP):
            n = g + b
            gp = b % NG
            sp = b % NSC
            wait_gather(n, gp)

            @pl.when(n >= NSC)
            def _():
                wait_scatter(n - NSC, sp)

            scale(gp, sp)
            fire_scatter(n, sp)

            @pl.when(n + NG < NCHUNK)
            def _():
                fire_gather(n + NG, gp)

    for m in range(NCHUNK - NSC, NCHUNK):
        wait_scatter(m, m % NSC)


def kernel(x, W):
    x_r = x.reshape(NW, NCHUNK, CHUNK).astype(jnp.int32)
    out = _embed(x_r, W)
    return out.reshape(B_SEQ, T_SEQ, D)
